# baseline (speedup 1.0000x reference)
import functools

import jax
import jax.numpy as jnp
from jax import lax
from jax.experimental import pallas as pl
from jax.experimental.pallas import tpu as pltpu
from jax.experimental.pallas import tpu_sc as plsc

_B, _S, _H = 4, 4096, 2048
_L = 16
_Q = 4
_C = _S // _Q


@functools.partial(
    pl.kernel,
    mesh=plsc.VectorSubcoreMesh(
        core_axis_name="c", subcore_axis_name="s", num_cores=1
    ),
    out_type=jax.ShapeDtypeStruct((_B, _H), jnp.float32),
    scratch_types=[
        pltpu.VMEM((_C,), jnp.int32),
        pltpu.VMEM((_L,), jnp.int32),
        pltpu.VMEM((_Q * _L,), jnp.int32),
        pltpu.VMEM((1, _H), jnp.float32),
        pltpu.VMEM_SHARED((_B * _Q * _L,), jnp.int32),
    ],
)
def _last_pool_sc(data_hbm, mask_hbm, out_hbm, mask_v, acc_v, parts_v,
                  row_v, shared):
    wid = lax.axis_index("s")
    b = wid // _Q
    q = wid % _Q

    pltpu.sync_copy(mask_hbm.at[pl.ds(b * _S + q * _C, _C)], mask_v)

    def body(i, acc):
        return acc + mask_v[pl.ds(i * _L, _L)]

    acc = lax.fori_loop(
        0, _C // _L, body, jnp.zeros((_L,), jnp.int32), unroll=8
    )
    acc_v[...] = acc
    pltpu.sync_copy(acc_v, shared.at[pl.ds(wid * _L, _L)])
    plsc.subcore_barrier()

    @pl.when(q == 0)
    def _():
        pltpu.sync_copy(shared.at[pl.ds(b * _Q * _L, _Q * _L)], parts_v)
        tot = parts_v[pl.ds(0, _L)]
        for k in range(1, _Q):
            tot = tot + parts_v[pl.ds(k * _L, _L)]
        length = tot[0]
        for j in range(1, _L):
            length = length + tot[j]
        t = jnp.where(length > 0, length - 1, _S - 1)
        pltpu.sync_copy(data_hbm.at[b, pl.ds(t, 1)], row_v)
        pltpu.sync_copy(row_v, out_hbm.at[pl.ds(b, 1)])


def kernel(data, padding_mask):
    return _last_pool_sc(data, padding_mask.astype(jnp.int32).reshape(-1))


# speedup vs baseline: 1.0747x; 1.0747x over previous
import functools

import jax
import jax.numpy as jnp
from jax import lax
from jax.experimental import pallas as pl
from jax.experimental.pallas import tpu as pltpu
from jax.experimental.pallas import tpu_sc as plsc

_B, _S, _H = 4, 4096, 2048
_L = 16


@functools.partial(
    pl.kernel,
    mesh=plsc.VectorSubcoreMesh(
        core_axis_name="c", subcore_axis_name="s", num_cores=1
    ),
    out_type=jax.ShapeDtypeStruct((_B, _H), jnp.float32),
    scratch_types=[
        pltpu.VMEM((_S,), jnp.int32),
        pltpu.VMEM((1, _H), jnp.float32),
        pltpu.SemaphoreType.DMA,
    ],
)
def _last_pool_sc(data_hbm, mask_hbm, out_hbm, mask_v, row_v, sem):
    wid = lax.axis_index("s")

    @pl.when(wid < _B)
    def _():
        b = wid
        spec = pltpu.async_copy(data_hbm.at[b, pl.ds(_S - 1, 1)], row_v, sem)
        pltpu.sync_copy(mask_hbm.at[pl.ds(b * _S, _S)], mask_v)

        def body(i, acc):
            return acc + mask_v[pl.ds(i * _L, _L)]

        acc = lax.fori_loop(
            0, _S // _L, body, jnp.zeros((_L,), jnp.int32), unroll=8
        )
        length = acc[0]
        for j in range(1, _L):
            length = length + acc[j]
        t = jnp.where(length > 0, length - 1, _S - 1)
        spec.wait()

        @pl.when(t != _S - 1)
        def _():
            pltpu.sync_copy(data_hbm.at[b, pl.ds(t, 1)], row_v)

        pltpu.sync_copy(row_v, out_hbm.at[pl.ds(b, 1)])


def kernel(data, padding_mask):
    return _last_pool_sc(data, padding_mask.astype(jnp.int32).reshape(-1))
